# baseline (speedup 1.0000x reference)
import functools

import jax
import jax.numpy as jnp
import numpy as np
from jax import lax
from jax.experimental import pallas as pl
from jax.experimental.pallas import tpu as pltpu
from jax.experimental.pallas import tpu_sc as plsc

N = 10000
E = 160000
DM = 256
H = 8
DH = 32
DFF = 1024
NC = 2
NS = 16
LANES = 16
NW = NC * NS
BB = 64
RPT = 80
EP = NW * RPT * BB
NPAD = EP - E
E16 = EP * LANES
NROW = EP // BB
NT = N + 16
ROWB = BB * LANES
INV_SQRT_DH = float(1.0 / np.sqrt(DH))

_mesh = lambda: plsc.VectorSubcoreMesh(core_axis_name="c", subcore_axis_name="s")


def _lane_perm(x, perm):
    return x.at[perm].get(mode="promise_in_bounds")


def _lane_max_splat(x):
    lane = lax.iota(jnp.int32, LANES)
    for k in (8, 4, 2, 1):
        x = jnp.maximum(x, _lane_perm(x, lane ^ k))
    return x


def _dot8_row(kr, qr, i):
    lane = lax.iota(jnp.int32, LANES)
    lt8 = lane < 8
    maskq = (lane & 4) == 0

    def rot(x, k):
        return _lane_perm(x, lane ^ k)

    p = []
    for h in range(H):
        a0 = kr[i, pl.ds(h * DH, LANES)] * qr[i, pl.ds(h * DH, LANES)]
        a1 = (kr[i, pl.ds(h * DH + LANES, LANES)]
              * qr[i, pl.ds(h * DH + LANES, LANES)])
        p.append(a0 + a1)
    m = []
    for a, b in ((0, 1), (2, 3), (4, 5), (6, 7)):
        m.append(jnp.where(lt8, p[a], p[b]) + rot(jnp.where(lt8, p[b], p[a]), 8))
    z = []
    for a, b in ((0, 1), (2, 3)):
        x2 = m[a] + rot(m[a], 4)
        y2 = m[b] + rot(m[b], 4)
        z.append(jnp.where(maskq, x2, rot(y2, 4)))
    zb = []
    for t in z:
        t = t + rot(t, 2)
        t = t + rot(t, 1)
        zb.append(t)
    fmap = ((lane & 1) << 3) | ((lane & 2) << 1)
    return jnp.where(lane < 4,
                     _lane_perm(zb[0], fmap),
                     _lane_perm(zb[1], fmap))


def _proj_block(qf_ref, kvf_ref, wq_ref, wk_ref, wv_ref,
                qh_ref, kh_ref, v0_ref, v1_ref):
    qh_ref[...] = jnp.dot(qf_ref[...], wq_ref[...],
                          preferred_element_type=jnp.float32)
    kh_ref[...] = jnp.dot(kvf_ref[...], wk_ref[...],
                          preferred_element_type=jnp.float32) * INV_SQRT_DH
    v = jnp.dot(kvf_ref[...], wv_ref[...], preferred_element_type=jnp.float32)
    v0_ref[...] = v[:, :128]
    v1_ref[...] = v[:, 128:]


def _projections(q_feat, kv_feat, Wq, Wk, Wv):
    BR = 2000
    full = lambda r, c: pl.BlockSpec((r, c), lambda i: (0, 0))
    row = lambda c: pl.BlockSpec((BR, c), lambda i: (i, 0))
    return pl.pallas_call(
        _proj_block,
        grid=(N // BR,),
        in_specs=[row(DM), row(DM), full(DM, DM), full(DM, DM), full(DM, DM)],
        out_specs=[row(DM), row(DM), row(128), row(128)],
        out_shape=[
            jax.ShapeDtypeStruct((N, DM), jnp.float32),
            jax.ShapeDtypeStruct((N, DM), jnp.float32),
            jax.ShapeDtypeStruct((N, 128), jnp.float32),
            jax.ShapeDtypeStruct((N, 128), jnp.float32),
        ],
    )(q_feat, kv_feat, Wq, Wk, Wv)


def _logits_body(kh, qh, src1d, dstg1d, e_out, tmax,
                 isall, idall, kr0, qr0, kr1, qr1, es0, es1, mstage,
                 sem0, sem1, esem0, esem1):
    c = lax.axis_index("c")
    s = lax.axis_index("s")
    wid = c * NS + s
    rbase = pl.multiple_of(wid * RPT, 8)
    lane = lax.iota(jnp.int32, LANES)
    neg = jnp.full((LANES,), -1e30, jnp.float32)
    lt8 = lane < 8

    pltpu.sync_copy(src1d.at[pl.ds(rbase * BB, RPT * BB)], isall)
    pltpu.sync_copy(dstg1d.at[pl.ds(rbase * BB, RPT * BB)], idall)

    def e_slice(r):
        return e_out.at[pl.ds((rbase + r) * ROWB, ROWB)]

    def gather(r, kr, qr, sem):
        ck = pltpu.async_copy(kh.at[isall.at[pl.ds(r * BB, BB)]], kr, sem)
        cq = pltpu.async_copy(qh.at[idall.at[pl.ds(r * BB, BB)]], qr, sem)
        return ck, cq

    def compute(kr, qr, es, mx):
        def edge2(ii, mx):
            for u in range(2):
                i = 2 * ii + u
                row = _dot8_row(kr, qr, i)
                es[pl.ds(i * LANES, LANES)] = row
                mx = jnp.maximum(mx, jnp.where(lt8, row, neg))
            return mx

        return lax.fori_loop(0, BB // 2, edge2, mx)

    NP = RPT // 2
    gather(0, kr0, qr0, sem0)
    gather(1, kr1, qr1, sem1)

    def pair(p, mx):
        r0 = 2 * p
        r1 = r0 + 1

        @pl.when(p > 0)
        def _():
            pltpu.make_async_copy(es0, e_slice(r0), esem0).wait()
            pltpu.make_async_copy(es1, e_slice(r1), esem1).wait()

        pltpu.make_async_copy(kh.at[isall.at[pl.ds(r0 * BB, BB)]], kr0,
                              sem0).wait()
        pltpu.make_async_copy(qh.at[idall.at[pl.ds(r0 * BB, BB)]], qr0,
                              sem0).wait()
        mx = compute(kr0, qr0, es0, mx)
        pltpu.async_copy(es0, e_slice(r0), esem0)

        @pl.when(p < NP - 1)
        def _():
            gather(r0 + 2, kr0, qr0, sem0)

        pltpu.make_async_copy(kh.at[isall.at[pl.ds(r1 * BB, BB)]], kr1,
                              sem1).wait()
        pltpu.make_async_copy(qh.at[idall.at[pl.ds(r1 * BB, BB)]], qr1,
                              sem1).wait()
        mx = compute(kr1, qr1, es1, mx)
        pltpu.async_copy(es1, e_slice(r1), esem1)

        @pl.when(p < NP - 1)
        def _():
            gather(r1 + 2, kr1, qr1, sem1)

        return mx

    mx = lax.fori_loop(0, NP, pair, neg)
    pltpu.make_async_copy(es0, e_slice(0), esem0).wait()
    pltpu.make_async_copy(es1, e_slice(1), esem1).wait()
    mstage[...] = mx
    pltpu.sync_copy(mstage, tmax.at[pl.ds(wid * LANES, LANES)])


def _edge_logits(kh, qh, src1d, dstg1d):
    f32 = jnp.float32
    kfn = pl.kernel(
        _logits_body,
        mesh=_mesh(),
        out_type=[
            jax.ShapeDtypeStruct((E16,), f32),
            jax.ShapeDtypeStruct((NW * LANES,), f32),
        ],
        scratch_types=[
            pltpu.VMEM((RPT * BB,), jnp.int32),
            pltpu.VMEM((RPT * BB,), jnp.int32),
            pltpu.VMEM((BB, DM), f32),
            pltpu.VMEM((BB, DM), f32),
            pltpu.VMEM((BB, DM), f32),
            pltpu.VMEM((BB, DM), f32),
            pltpu.VMEM((ROWB,), f32),
            pltpu.VMEM((ROWB,), f32),
            pltpu.VMEM((LANES,), f32),
            pltpu.SemaphoreType.DMA,
            pltpu.SemaphoreType.DMA,
            pltpu.SemaphoreType.DMA,
            pltpu.SemaphoreType.DMA,
        ],
    )
    return kfn(kh, qh, src1d, dstg1d)


def _load_m(tbuf):
    m = tbuf[pl.ds(0, LANES)]
    for i in range(1, NW):
        m = jnp.maximum(m, tbuf[pl.ds(i * LANES, LANES)])
    return _lane_max_splat(m)


def _denom_body(e_in, dsts2d, tmax, zs, s0, s1,
                shared_s, eb0, eb1, wb0, wb1, idall, tbuf,
                sem0, sem1, ssem0, ssem1):
    c = lax.axis_index("c")
    s = lax.axis_index("s")
    wid = c * NS + s
    rbase = pl.multiple_of(wid * RPT, 8)
    lane = lax.iota(jnp.int32, LANES)

    @pl.when(s == 0)
    def _():
        pltpu.sync_copy(zs, shared_s)

    pltpu.sync_copy(zs.at[pl.ds(0, BB)], wb0)
    pltpu.sync_copy(zs.at[pl.ds(0, BB)], wb1)
    pltpu.sync_copy(dsts2d.at[pl.ds(rbase, RPT)], idall)
    pltpu.sync_copy(tmax, tbuf)
    M = _load_m(tbuf)
    plsc.subcore_barrier()

    def e_slice(r):
        return e_in.at[pl.ds((rbase + r) * ROWB, ROWB)]

    def compute(eb, wb):
        def rowf(ii, carry):
            for u in range(4):
                i = 4 * ii + u
                r = eb[pl.ds(i * LANES, LANES)]
                wb[i, pl.ds(0, LANES)] = jnp.where(lane < H,
                                                   jnp.exp(r - M), 0.0)
            return carry

        lax.fori_loop(0, BB // 4, rowf, 0)

    NP = RPT // 2
    pltpu.async_copy(e_slice(0), eb0, sem0)
    pltpu.async_copy(e_slice(1), eb1, sem1)

    def pair(p, carry):
        r0 = 2 * p
        r1 = r0 + 1

        @pl.when(p > 0)
        def _():
            pltpu.make_async_copy(wb0, shared_s.at[idall.at[r0]], ssem0).wait()
            pltpu.make_async_copy(wb1, shared_s.at[idall.at[r1]], ssem1).wait()

        pltpu.make_async_copy(e_slice(r0), eb0, sem0).wait()
        compute(eb0, wb0)
        pltpu.async_copy(wb0, shared_s.at[idall.at[r0]], ssem0, add=True)

        @pl.when(p < NP - 1)
        def _():
            pltpu.async_copy(e_slice(r0 + 2), eb0, sem0)

        pltpu.make_async_copy(e_slice(r1), eb1, sem1).wait()
        compute(eb1, wb1)
        pltpu.async_copy(wb1, shared_s.at[idall.at[r1]], ssem1, add=True)

        @pl.when(p < NP - 1)
        def _():
            pltpu.async_copy(e_slice(r1 + 2), eb1, sem1)

        return carry

    lax.fori_loop(0, NP, pair, 0)
    pltpu.make_async_copy(wb0, shared_s.at[idall.at[0]], ssem0).wait()
    pltpu.make_async_copy(wb1, shared_s.at[idall.at[1]], ssem1).wait()
    plsc.subcore_barrier()

    rows = 1000
    off = pl.multiple_of(s * rows, 8)

    @pl.when(jnp.logical_and(c == 0, s < N // rows))
    def _():
        pltpu.sync_copy(shared_s.at[pl.ds(off, rows)],
                        s0.at[pl.ds(off, rows)])

    @pl.when(jnp.logical_and(c == 1, s < N // rows))
    def _():
        pltpu.sync_copy(shared_s.at[pl.ds(off, rows)],
                        s1.at[pl.ds(off, rows)])


def _edge_denoms(e_arr, dsts2d, tmax, zs):
    f32 = jnp.float32
    kfn = pl.kernel(
        _denom_body,
        mesh=_mesh(),
        out_type=[
            jax.ShapeDtypeStruct((N, 128), f32),
            jax.ShapeDtypeStruct((N, 128), f32),
        ],
        scratch_types=[
            pltpu.VMEM_SHARED((NT, 128), f32),
            pltpu.VMEM((ROWB,), f32),
            pltpu.VMEM((ROWB,), f32),
            pltpu.VMEM((BB, 128), f32),
            pltpu.VMEM((BB, 128), f32),
            pltpu.VMEM((RPT, BB), jnp.int32),
            pltpu.VMEM((NW * LANES,), f32),
            pltpu.SemaphoreType.DMA,
            pltpu.SemaphoreType.DMA,
            pltpu.SemaphoreType.DMA,
            pltpu.SemaphoreType.DMA,
        ],
    )
    return kfn(e_arr, dsts2d, tmax, zs)


def _agg_body(g, vg, e_in, tmax, src1d, dsts2d, zs, og0, og1,
              shared_o, vb0, vb1, mb0, mb1, eb0, eb1, isall, idall, tbuf,
              sem0, sem1, ssem0, ssem1):
    c = lax.axis_index("c")
    s = lax.axis_index("s")
    wid = c * NS + s
    rbase = pl.multiple_of(wid * RPT, 8)

    @pl.when(s == 0)
    def _():
        pltpu.sync_copy(zs, shared_o)

    pltpu.sync_copy(src1d.at[pl.ds(rbase * BB, RPT * BB)], isall)
    pltpu.sync_copy(dsts2d.at[pl.ds(rbase, RPT)], idall)
    pltpu.sync_copy(tmax, tbuf)
    M = _load_m(tbuf)
    plsc.subcore_barrier()

    def e_slice(r):
        return e_in.at[pl.ds((rbase + r) * ROWB, ROWB)]

    def compute(vb, eb, mb):
        def edge(ii, carry):
            for u in range(2):
                i = 2 * ii + u
                arow = jnp.exp(eb[pl.ds(i * LANES, LANES)] - M)
                for hh in range(4):
                    hsel = jnp.full((LANES,), g * 4 + hh, jnp.int32)
                    spl = arow.at[hsel].get(mode="promise_in_bounds")
                    lo = vb[i, pl.ds(hh * DH, LANES)] * spl
                    hi = vb[i, pl.ds(hh * DH + LANES, LANES)] * spl
                    mb[i, pl.ds(hh * DH, LANES)] = lo
                    mb[i, pl.ds(hh * DH + LANES, LANES)] = hi
            return carry

        lax.fori_loop(0, BB // 2, edge, 0)

    def gathers(r, vb, eb, sem):
        pltpu.async_copy(vg.at[isall.at[pl.ds(r * BB, BB)]], vb, sem)
        pltpu.async_copy(e_slice(r), eb, sem)

    NP = RPT // 2
    gathers(0, vb0, eb0, sem0)
    gathers(1, vb1, eb1, sem1)

    def pair(p, carry):
        r0 = 2 * p
        r1 = r0 + 1

        @pl.when(p > 0)
        def _():
            pltpu.make_async_copy(mb0, shared_o.at[idall.at[r0]], ssem0).wait()
            pltpu.make_async_copy(mb1, shared_o.at[idall.at[r1]], ssem1).wait()

        pltpu.make_async_copy(vg.at[isall.at[pl.ds(r0 * BB, BB)]], vb0,
                              sem0).wait()
        pltpu.make_async_copy(e_slice(r0), eb0, sem0).wait()
        compute(vb0, eb0, mb0)
        pltpu.async_copy(mb0, shared_o.at[idall.at[r0]], ssem0, add=True)

        @pl.when(p < NP - 1)
        def _():
            gathers(r0 + 2, vb0, eb0, sem0)

        pltpu.make_async_copy(vg.at[isall.at[pl.ds(r1 * BB, BB)]], vb1,
                              sem1).wait()
        pltpu.make_async_copy(e_slice(r1), eb1, sem1).wait()
        compute(vb1, eb1, mb1)
        pltpu.async_copy(mb1, shared_o.at[idall.at[r1]], ssem1, add=True)

        @pl.when(p < NP - 1)
        def _():
            gathers(r1 + 2, vb1, eb1, sem1)

        return carry

    lax.fori_loop(0, NP, pair, 0)
    pltpu.make_async_copy(mb0, shared_o.at[idall.at[0]], ssem0).wait()
    pltpu.make_async_copy(mb1, shared_o.at[idall.at[1]], ssem1).wait()
    plsc.subcore_barrier()

    rows = 1000
    off2 = pl.multiple_of(s * rows, 8)

    @pl.when(jnp.logical_and(c == 0, s < N // rows))
    def _():
        pltpu.sync_copy(shared_o.at[pl.ds(off2, rows)],
                        og0.at[pl.ds(off2, rows)])

    @pl.when(jnp.logical_and(c == 1, s < N // rows))
    def _():
        pltpu.sync_copy(shared_o.at[pl.ds(off2, rows)],
                        og1.at[pl.ds(off2, rows)])


def _aggregate(g, vg, e_arr, tmax, src1d, dsts2d, zs):
    f32 = jnp.float32
    kfn = pl.kernel(
        functools.partial(_agg_body, g),
        mesh=_mesh(),
        out_type=[
            jax.ShapeDtypeStruct((N, 128), f32),
            jax.ShapeDtypeStruct((N, 128), f32),
        ],
        scratch_types=[
            pltpu.VMEM_SHARED((NT, 128), f32),
            pltpu.VMEM((BB, 128), f32),
            pltpu.VMEM((BB, 128), f32),
            pltpu.VMEM((BB, 128), f32),
            pltpu.VMEM((BB, 128), f32),
            pltpu.VMEM((ROWB,), f32),
            pltpu.VMEM((ROWB,), f32),
            pltpu.VMEM((RPT * BB,), jnp.int32),
            pltpu.VMEM((RPT, BB), jnp.int32),
            pltpu.VMEM((NW * LANES,), f32),
            pltpu.SemaphoreType.DMA,
            pltpu.SemaphoreType.DMA,
            pltpu.SemaphoreType.DMA,
            pltpu.SemaphoreType.DMA,
        ],
    )
    return kfn(vg, e_arr, tmax, src1d, dsts2d, zs)


def _post_block(o00_ref, o01_ref, o10_ref, o11_ref, s0_ref, s1_ref,
                qf_ref, wo_ref, w1_ref, b1_ref, w2_ref, b2_ref,
                gin_ref, bin_ref, gint_ref, bint_ref, out_ref):
    a0 = o00_ref[...] + o01_ref[...]
    a1 = o10_ref[...] + o11_ref[...]
    sden = s0_ref[...] + s1_ref[...] + 1e-9
    br = a0.shape[0]
    rep = jnp.concatenate(
        [jnp.broadcast_to(sden[:, h:h + 1], (br, DH)) for h in range(H)],
        axis=1)
    a0 = a0 / rep[:, :128]
    a1 = a1 / rep[:, 128:]
    attn = jnp.concatenate([a0, a1], axis=1)
    sa = jnp.dot(attn, wo_ref[...], preferred_element_type=jnp.float32)
    x = qf_ref[...] + sa
    mu = jnp.mean(x, axis=-1, keepdims=True)
    var = jnp.mean((x - mu) ** 2, axis=-1, keepdims=True)
    x = (x - mu) / jnp.sqrt(var + 1e-5) * gin_ref[...] + bin_ref[...]
    hmid = jnp.maximum(jnp.dot(x, w1_ref[...], preferred_element_type=jnp.float32)
                       + b1_ref[...], 0.0)
    f = jnp.dot(hmid, w2_ref[...], preferred_element_type=jnp.float32) + b2_ref[...]
    y = x + f
    mu2 = jnp.mean(y, axis=-1, keepdims=True)
    var2 = jnp.mean((y - mu2) ** 2, axis=-1, keepdims=True)
    out_ref[...] = ((y - mu2) / jnp.sqrt(var2 + 1e-5) * gint_ref[...]
                    + bint_ref[...])


def _post(o00, o01, o10, o11, s0, s1, q_feat, Wo, W1, b1, W2, b2,
          g_in, b_in, g_inter, b_inter):
    BR = 1000
    full = lambda r, c: pl.BlockSpec((r, c), lambda i: (0, 0))
    row = lambda c: pl.BlockSpec((BR, c), lambda i: (i, 0))
    return pl.pallas_call(
        _post_block,
        grid=(N // BR,),
        in_specs=[row(128), row(128), row(128), row(128),
                  row(128), row(128), row(DM),
                  full(DM, DM), full(DM, DFF), full(1, DFF),
                  full(DFF, DM), full(1, DM), full(1, DM), full(1, DM),
                  full(1, DM), full(1, DM)],
        out_specs=row(DM),
        out_shape=jax.ShapeDtypeStruct((N, DM), jnp.float32),
    )(o00, o01, o10, o11, s0, s1, q_feat, Wo, W1, b1.reshape(1, -1), W2,
      b2.reshape(1, -1), g_in.reshape(1, -1), b_in.reshape(1, -1),
      g_inter.reshape(1, -1), b_inter.reshape(1, -1))


def kernel(q_feat, kv_feat, edge_index, q_nids, kv_nids,
           Wq, Wk, Wv, Wo, W1, b1, W2, b2, g_in, b_in, g_inter, b_inter):
    src = edge_index[0]
    dst = edge_index[1]
    idt = src.dtype
    padi = jnp.arange(NPAD, dtype=idt)
    src1d = jnp.concatenate([src, padi % N])
    dstg1d = jnp.concatenate([dst, padi % N])
    dsts2d = jnp.concatenate([dst, N + padi % (NT - N)]).reshape(NROW, BB)
    qh, kh, v0, v1 = _projections(q_feat, kv_feat, Wq, Wk, Wv)
    zs = jnp.zeros((NT, 128), jnp.float32)
    e_arr, tmax = _edge_logits(kh, qh, src1d, dstg1d)
    s0, s1 = _edge_denoms(e_arr, dsts2d, tmax, zs)
    o00, o01 = _aggregate(0, v0, e_arr, tmax, src1d, dsts2d, zs)
    o10, o11 = _aggregate(1, v1, e_arr, tmax, src1d, dsts2d, zs)
    return _post(o00, o01, o10, o11, s0, s1, q_feat, Wo, W1, b1, W2, b2,
                 g_in, b_in, g_inter, b_inter)


# speedup vs baseline: 1.0143x; 1.0143x over previous
import functools

import jax
import jax.numpy as jnp
import numpy as np
from jax import lax
from jax.experimental import pallas as pl
from jax.experimental.pallas import tpu as pltpu
from jax.experimental.pallas import tpu_sc as plsc

N = 10000
E = 160000
DM = 256
H = 8
DH = 32
DFF = 1024
NC = 2
NS = 16
LANES = 16
NW = NC * NS
BB = 64
RPT = 80
EP = NW * RPT * BB
NPAD = EP - E
E16 = EP * LANES
NROW = EP // BB
NT = N + 16
ROWB = BB * LANES
INV_SQRT_DH = float(1.0 / np.sqrt(DH))

_mesh = lambda: plsc.VectorSubcoreMesh(core_axis_name="c", subcore_axis_name="s")


def _lane_perm(x, perm):
    return x.at[perm].get(mode="promise_in_bounds")


def _lane_max_splat(x):
    lane = lax.iota(jnp.int32, LANES)
    for k in (8, 4, 2, 1):
        x = jnp.maximum(x, _lane_perm(x, lane ^ k))
    return x


def _dot8_row(kr, qr, i):
    lane = lax.iota(jnp.int32, LANES)
    lt8 = lane < 8
    maskq = (lane & 4) == 0

    def rot(x, k):
        return _lane_perm(x, lane ^ k)

    p = []
    for h in range(H):
        a0 = kr[i, pl.ds(h * DH, LANES)] * qr[i, pl.ds(h * DH, LANES)]
        a1 = (kr[i, pl.ds(h * DH + LANES, LANES)]
              * qr[i, pl.ds(h * DH + LANES, LANES)])
        p.append(a0 + a1)
    m = []
    for a, b in ((0, 1), (2, 3), (4, 5), (6, 7)):
        m.append(jnp.where(lt8, p[a], p[b]) + rot(jnp.where(lt8, p[b], p[a]), 8))
    z = []
    for a, b in ((0, 1), (2, 3)):
        x2 = m[a] + rot(m[a], 4)
        y2 = m[b] + rot(m[b], 4)
        z.append(jnp.where(maskq, x2, rot(y2, 4)))
    zb = []
    for t in z:
        t = t + rot(t, 2)
        t = t + rot(t, 1)
        zb.append(t)
    fmap = ((lane & 1) << 3) | ((lane & 2) << 1)
    return jnp.where(lane < 4,
                     _lane_perm(zb[0], fmap),
                     _lane_perm(zb[1], fmap))


def _proj_block(qf_ref, kvf_ref, wq_ref, wk_ref, wv_ref,
                qh_ref, kh_ref, v0_ref, v1_ref):
    qh_ref[...] = jnp.dot(qf_ref[...], wq_ref[...],
                          preferred_element_type=jnp.float32)
    kh_ref[...] = jnp.dot(kvf_ref[...], wk_ref[...],
                          preferred_element_type=jnp.float32) * INV_SQRT_DH
    v = jnp.dot(kvf_ref[...], wv_ref[...], preferred_element_type=jnp.float32)
    v0_ref[...] = v[:, :128]
    v1_ref[...] = v[:, 128:]


def _projections(q_feat, kv_feat, Wq, Wk, Wv):
    BR = 2000
    full = lambda r, c: pl.BlockSpec((r, c), lambda i: (0, 0))
    row = lambda c: pl.BlockSpec((BR, c), lambda i: (i, 0))
    return pl.pallas_call(
        _proj_block,
        grid=(N // BR,),
        in_specs=[row(DM), row(DM), full(DM, DM), full(DM, DM), full(DM, DM)],
        out_specs=[row(DM), row(DM), row(128), row(128)],
        out_shape=[
            jax.ShapeDtypeStruct((N, DM), jnp.float32),
            jax.ShapeDtypeStruct((N, DM), jnp.float32),
            jax.ShapeDtypeStruct((N, 128), jnp.float32),
            jax.ShapeDtypeStruct((N, 128), jnp.float32),
        ],
    )(q_feat, kv_feat, Wq, Wk, Wv)


def _logits_body(kh, qh, src1d, dstg1d, e_out, tmax,
                 isall, idall, kr0, qr0, kr1, qr1, es0, es1, mstage,
                 sem0, sem1, esem0, esem1):
    c = lax.axis_index("c")
    s = lax.axis_index("s")
    wid = c * NS + s
    rbase = pl.multiple_of(wid * RPT, 8)
    lane = lax.iota(jnp.int32, LANES)
    neg = jnp.full((LANES,), -1e30, jnp.float32)
    lt8 = lane < 8

    pltpu.sync_copy(src1d.at[pl.ds(rbase * BB, RPT * BB)], isall)
    pltpu.sync_copy(dstg1d.at[pl.ds(rbase * BB, RPT * BB)], idall)

    def e_slice(r):
        return e_out.at[pl.ds((rbase + r) * ROWB, ROWB)]

    def gather(r, kr, qr, sem):
        ck = pltpu.async_copy(kh.at[isall.at[pl.ds(r * BB, BB)]], kr, sem)
        cq = pltpu.async_copy(qh.at[idall.at[pl.ds(r * BB, BB)]], qr, sem)
        return ck, cq

    def compute(kr, qr, es, mx):
        def edge2(ii, mx):
            for u in range(2):
                i = 2 * ii + u
                row = _dot8_row(kr, qr, i)
                es[pl.ds(i * LANES, LANES)] = row
                mx = jnp.maximum(mx, jnp.where(lt8, row, neg))
            return mx

        return lax.fori_loop(0, BB // 2, edge2, mx)

    NP = RPT // 2
    gather(0, kr0, qr0, sem0)
    gather(1, kr1, qr1, sem1)

    def pair(p, mx):
        r0 = 2 * p
        r1 = r0 + 1

        @pl.when(p > 0)
        def _():
            pltpu.make_async_copy(es0, e_slice(r0), esem0).wait()
            pltpu.make_async_copy(es1, e_slice(r1), esem1).wait()

        pltpu.make_async_copy(kh.at[isall.at[pl.ds(r0 * BB, BB)]], kr0,
                              sem0).wait()
        pltpu.make_async_copy(qh.at[idall.at[pl.ds(r0 * BB, BB)]], qr0,
                              sem0).wait()
        mx = compute(kr0, qr0, es0, mx)
        pltpu.async_copy(es0, e_slice(r0), esem0)

        @pl.when(p < NP - 1)
        def _():
            gather(r0 + 2, kr0, qr0, sem0)

        pltpu.make_async_copy(kh.at[isall.at[pl.ds(r1 * BB, BB)]], kr1,
                              sem1).wait()
        pltpu.make_async_copy(qh.at[idall.at[pl.ds(r1 * BB, BB)]], qr1,
                              sem1).wait()
        mx = compute(kr1, qr1, es1, mx)
        pltpu.async_copy(es1, e_slice(r1), esem1)

        @pl.when(p < NP - 1)
        def _():
            gather(r1 + 2, kr1, qr1, sem1)

        return mx

    mx = lax.fori_loop(0, NP, pair, neg)
    pltpu.make_async_copy(es0, e_slice(0), esem0).wait()
    pltpu.make_async_copy(es1, e_slice(1), esem1).wait()
    mstage[...] = mx
    pltpu.sync_copy(mstage, tmax.at[pl.ds(wid * LANES, LANES)])


def _edge_logits(kh, qh, src1d, dstg1d):
    f32 = jnp.float32
    kfn = pl.kernel(
        _logits_body,
        mesh=_mesh(),
        out_type=[
            jax.ShapeDtypeStruct((E16,), f32),
            jax.ShapeDtypeStruct((NW * LANES,), f32),
        ],
        scratch_types=[
            pltpu.VMEM((RPT * BB,), jnp.int32),
            pltpu.VMEM((RPT * BB,), jnp.int32),
            pltpu.VMEM((BB, DM), f32),
            pltpu.VMEM((BB, DM), f32),
            pltpu.VMEM((BB, DM), f32),
            pltpu.VMEM((BB, DM), f32),
            pltpu.VMEM((ROWB,), f32),
            pltpu.VMEM((ROWB,), f32),
            pltpu.VMEM((LANES,), f32),
            pltpu.SemaphoreType.DMA,
            pltpu.SemaphoreType.DMA,
            pltpu.SemaphoreType.DMA,
            pltpu.SemaphoreType.DMA,
        ],
    )
    return kfn(kh, qh, src1d, dstg1d)


def _load_m(tbuf):
    m = tbuf[pl.ds(0, LANES)]
    for i in range(1, NW):
        m = jnp.maximum(m, tbuf[pl.ds(i * LANES, LANES)])
    return _lane_max_splat(m)


def _denom_body(e_in, dsts2d, tmax, zs, s0, s1,
                shared_s, eb0, eb1, wb0, wb1, idall, tbuf,
                sem0, sem1, ssem0, ssem1):
    c = lax.axis_index("c")
    s = lax.axis_index("s")
    wid = c * NS + s
    rbase = pl.multiple_of(wid * RPT, 8)
    lane = lax.iota(jnp.int32, LANES)

    @pl.when(s == 0)
    def _():
        pltpu.sync_copy(zs, shared_s)

    pltpu.sync_copy(zs.at[pl.ds(0, BB)], wb0)
    pltpu.sync_copy(zs.at[pl.ds(0, BB)], wb1)
    pltpu.sync_copy(dsts2d.at[pl.ds(rbase, RPT)], idall)
    pltpu.sync_copy(tmax, tbuf)
    M = _load_m(tbuf)
    plsc.subcore_barrier()

    def e_slice(r):
        return e_in.at[pl.ds((rbase + r) * ROWB, ROWB)]

    def compute(eb, wb):
        def rowf(i, carry):
            r = eb[pl.ds(i * LANES, LANES)]
            wb[i, pl.ds(0, LANES)] = jnp.where(lane < H, jnp.exp(r - M), 0.0)
            return carry

        lax.fori_loop(0, BB, rowf, 0)

    NP = RPT // 2
    pltpu.async_copy(e_slice(0), eb0, sem0)
    pltpu.async_copy(e_slice(1), eb1, sem1)

    def pair(p, carry):
        r0 = 2 * p
        r1 = r0 + 1

        @pl.when(p > 0)
        def _():
            pltpu.make_async_copy(wb0, shared_s.at[idall.at[r0]], ssem0).wait()
            pltpu.make_async_copy(wb1, shared_s.at[idall.at[r1]], ssem1).wait()

        pltpu.make_async_copy(e_slice(r0), eb0, sem0).wait()
        compute(eb0, wb0)
        pltpu.async_copy(wb0, shared_s.at[idall.at[r0]], ssem0, add=True)

        @pl.when(p < NP - 1)
        def _():
            pltpu.async_copy(e_slice(r0 + 2), eb0, sem0)

        pltpu.make_async_copy(e_slice(r1), eb1, sem1).wait()
        compute(eb1, wb1)
        pltpu.async_copy(wb1, shared_s.at[idall.at[r1]], ssem1, add=True)

        @pl.when(p < NP - 1)
        def _():
            pltpu.async_copy(e_slice(r1 + 2), eb1, sem1)

        return carry

    lax.fori_loop(0, NP, pair, 0)
    pltpu.make_async_copy(wb0, shared_s.at[idall.at[0]], ssem0).wait()
    pltpu.make_async_copy(wb1, shared_s.at[idall.at[1]], ssem1).wait()
    plsc.subcore_barrier()

    rows = 1000
    off = pl.multiple_of(s * rows, 8)

    @pl.when(jnp.logical_and(c == 0, s < N // rows))
    def _():
        pltpu.sync_copy(shared_s.at[pl.ds(off, rows)],
                        s0.at[pl.ds(off, rows)])

    @pl.when(jnp.logical_and(c == 1, s < N // rows))
    def _():
        pltpu.sync_copy(shared_s.at[pl.ds(off, rows)],
                        s1.at[pl.ds(off, rows)])


def _edge_denoms(e_arr, dsts2d, tmax, zs):
    f32 = jnp.float32
    kfn = pl.kernel(
        _denom_body,
        mesh=_mesh(),
        out_type=[
            jax.ShapeDtypeStruct((N, 128), f32),
            jax.ShapeDtypeStruct((N, 128), f32),
        ],
        scratch_types=[
            pltpu.VMEM_SHARED((NT, 128), f32),
            pltpu.VMEM((ROWB,), f32),
            pltpu.VMEM((ROWB,), f32),
            pltpu.VMEM((BB, 128), f32),
            pltpu.VMEM((BB, 128), f32),
            pltpu.VMEM((RPT, BB), jnp.int32),
            pltpu.VMEM((NW * LANES,), f32),
            pltpu.SemaphoreType.DMA,
            pltpu.SemaphoreType.DMA,
            pltpu.SemaphoreType.DMA,
            pltpu.SemaphoreType.DMA,
        ],
    )
    return kfn(e_arr, dsts2d, tmax, zs)


def _agg_body(g, vg, e_in, tmax, src1d, dsts2d, zs, og0, og1,
              shared_o, vb0, vb1, mb0, mb1, eb0, eb1, isall, idall, tbuf,
              sem0, sem1, ssem0, ssem1):
    c = lax.axis_index("c")
    s = lax.axis_index("s")
    wid = c * NS + s
    rbase = pl.multiple_of(wid * RPT, 8)

    @pl.when(s == 0)
    def _():
        pltpu.sync_copy(zs, shared_o)

    pltpu.sync_copy(src1d.at[pl.ds(rbase * BB, RPT * BB)], isall)
    pltpu.sync_copy(dsts2d.at[pl.ds(rbase, RPT)], idall)
    pltpu.sync_copy(tmax, tbuf)
    M = _load_m(tbuf)
    plsc.subcore_barrier()

    def e_slice(r):
        return e_in.at[pl.ds((rbase + r) * ROWB, ROWB)]

    def compute(vb, eb, mb):
        def edge(i, carry):
            arow = jnp.exp(eb[pl.ds(i * LANES, LANES)] - M)
            for hh in range(4):
                hsel = jnp.full((LANES,), g * 4 + hh, jnp.int32)
                spl = arow.at[hsel].get(mode="promise_in_bounds")
                lo = vb[i, pl.ds(hh * DH, LANES)] * spl
                hi = vb[i, pl.ds(hh * DH + LANES, LANES)] * spl
                mb[i, pl.ds(hh * DH, LANES)] = lo
                mb[i, pl.ds(hh * DH + LANES, LANES)] = hi
            return carry

        lax.fori_loop(0, BB, edge, 0)

    def gathers(r, vb, eb, sem):
        pltpu.async_copy(vg.at[isall.at[pl.ds(r * BB, BB)]], vb, sem)
        pltpu.async_copy(e_slice(r), eb, sem)

    NP = RPT // 2
    gathers(0, vb0, eb0, sem0)
    gathers(1, vb1, eb1, sem1)

    def pair(p, carry):
        r0 = 2 * p
        r1 = r0 + 1

        @pl.when(p > 0)
        def _():
            pltpu.make_async_copy(mb0, shared_o.at[idall.at[r0]], ssem0).wait()
            pltpu.make_async_copy(mb1, shared_o.at[idall.at[r1]], ssem1).wait()

        pltpu.make_async_copy(vg.at[isall.at[pl.ds(r0 * BB, BB)]], vb0,
                              sem0).wait()
        pltpu.make_async_copy(e_slice(r0), eb0, sem0).wait()
        compute(vb0, eb0, mb0)
        pltpu.async_copy(mb0, shared_o.at[idall.at[r0]], ssem0, add=True)

        @pl.when(p < NP - 1)
        def _():
            gathers(r0 + 2, vb0, eb0, sem0)

        pltpu.make_async_copy(vg.at[isall.at[pl.ds(r1 * BB, BB)]], vb1,
                              sem1).wait()
        pltpu.make_async_copy(e_slice(r1), eb1, sem1).wait()
        compute(vb1, eb1, mb1)
        pltpu.async_copy(mb1, shared_o.at[idall.at[r1]], ssem1, add=True)

        @pl.when(p < NP - 1)
        def _():
            gathers(r1 + 2, vb1, eb1, sem1)

        return carry

    lax.fori_loop(0, NP, pair, 0)
    pltpu.make_async_copy(mb0, shared_o.at[idall.at[0]], ssem0).wait()
    pltpu.make_async_copy(mb1, shared_o.at[idall.at[1]], ssem1).wait()
    plsc.subcore_barrier()

    rows = 1000
    off2 = pl.multiple_of(s * rows, 8)

    @pl.when(jnp.logical_and(c == 0, s < N // rows))
    def _():
        pltpu.sync_copy(shared_o.at[pl.ds(off2, rows)],
                        og0.at[pl.ds(off2, rows)])

    @pl.when(jnp.logical_and(c == 1, s < N // rows))
    def _():
        pltpu.sync_copy(shared_o.at[pl.ds(off2, rows)],
                        og1.at[pl.ds(off2, rows)])


def _aggregate(g, vg, e_arr, tmax, src1d, dsts2d, zs):
    f32 = jnp.float32
    kfn = pl.kernel(
        functools.partial(_agg_body, g),
        mesh=_mesh(),
        out_type=[
            jax.ShapeDtypeStruct((N, 128), f32),
            jax.ShapeDtypeStruct((N, 128), f32),
        ],
        scratch_types=[
            pltpu.VMEM_SHARED((NT, 128), f32),
            pltpu.VMEM((BB, 128), f32),
            pltpu.VMEM((BB, 128), f32),
            pltpu.VMEM((BB, 128), f32),
            pltpu.VMEM((BB, 128), f32),
            pltpu.VMEM((ROWB,), f32),
            pltpu.VMEM((ROWB,), f32),
            pltpu.VMEM((RPT * BB,), jnp.int32),
            pltpu.VMEM((RPT, BB), jnp.int32),
            pltpu.VMEM((NW * LANES,), f32),
            pltpu.SemaphoreType.DMA,
            pltpu.SemaphoreType.DMA,
            pltpu.SemaphoreType.DMA,
            pltpu.SemaphoreType.DMA,
        ],
    )
    return kfn(vg, e_arr, tmax, src1d, dsts2d, zs)


def _post_block(o00_ref, o01_ref, o10_ref, o11_ref, s0_ref, s1_ref,
                qf_ref, wo_ref, w1_ref, b1_ref, w2_ref, b2_ref,
                gin_ref, bin_ref, gint_ref, bint_ref, out_ref):
    a0 = o00_ref[...] + o01_ref[...]
    a1 = o10_ref[...] + o11_ref[...]
    sden = s0_ref[...] + s1_ref[...] + 1e-9
    br = a0.shape[0]
    rep = jnp.concatenate(
        [jnp.broadcast_to(sden[:, h:h + 1], (br, DH)) for h in range(H)],
        axis=1)
    a0 = a0 / rep[:, :128]
    a1 = a1 / rep[:, 128:]
    attn = jnp.concatenate([a0, a1], axis=1)
    sa = jnp.dot(attn, wo_ref[...], preferred_element_type=jnp.float32)
    x = qf_ref[...] + sa
    mu = jnp.mean(x, axis=-1, keepdims=True)
    var = jnp.mean((x - mu) ** 2, axis=-1, keepdims=True)
    x = (x - mu) / jnp.sqrt(var + 1e-5) * gin_ref[...] + bin_ref[...]
    hmid = jnp.maximum(jnp.dot(x, w1_ref[...], preferred_element_type=jnp.float32)
                       + b1_ref[...], 0.0)
    f = jnp.dot(hmid, w2_ref[...], preferred_element_type=jnp.float32) + b2_ref[...]
    y = x + f
    mu2 = jnp.mean(y, axis=-1, keepdims=True)
    var2 = jnp.mean((y - mu2) ** 2, axis=-1, keepdims=True)
    out_ref[...] = ((y - mu2) / jnp.sqrt(var2 + 1e-5) * gint_ref[...]
                    + bint_ref[...])


def _post(o00, o01, o10, o11, s0, s1, q_feat, Wo, W1, b1, W2, b2,
          g_in, b_in, g_inter, b_inter):
    BR = 1000
    full = lambda r, c: pl.BlockSpec((r, c), lambda i: (0, 0))
    row = lambda c: pl.BlockSpec((BR, c), lambda i: (i, 0))
    return pl.pallas_call(
        _post_block,
        grid=(N // BR,),
        in_specs=[row(128), row(128), row(128), row(128),
                  row(128), row(128), row(DM),
                  full(DM, DM), full(DM, DFF), full(1, DFF),
                  full(DFF, DM), full(1, DM), full(1, DM), full(1, DM),
                  full(1, DM), full(1, DM)],
        out_specs=row(DM),
        out_shape=jax.ShapeDtypeStruct((N, DM), jnp.float32),
    )(o00, o01, o10, o11, s0, s1, q_feat, Wo, W1, b1.reshape(1, -1), W2,
      b2.reshape(1, -1), g_in.reshape(1, -1), b_in.reshape(1, -1),
      g_inter.reshape(1, -1), b_inter.reshape(1, -1))


def kernel(q_feat, kv_feat, edge_index, q_nids, kv_nids,
           Wq, Wk, Wv, Wo, W1, b1, W2, b2, g_in, b_in, g_inter, b_inter):
    src = edge_index[0]
    dst = edge_index[1]
    idt = src.dtype
    padi = jnp.arange(NPAD, dtype=idt)
    src1d = jnp.concatenate([src, padi % N])
    dstg1d = jnp.concatenate([dst, padi % N])
    dsts2d = jnp.concatenate([dst, N + padi % (NT - N)]).reshape(NROW, BB)
    qh, kh, v0, v1 = _projections(q_feat, kv_feat, Wq, Wk, Wv)
    zs = jnp.zeros((NT, 128), jnp.float32)
    e_arr, tmax = _edge_logits(kh, qh, src1d, dstg1d)
    s0, s1 = _edge_denoms(e_arr, dsts2d, tmax, zs)
    o00, o01 = _aggregate(0, v0, e_arr, tmax, src1d, dsts2d, zs)
    o10, o11 = _aggregate(1, v1, e_arr, tmax, src1d, dsts2d, zs)
    return _post(o00, o01, o10, o11, s0, s1, q_feat, Wo, W1, b1, W2, b2,
                 g_in, b_in, g_inter, b_inter)
